# baseline (speedup 1.0000x reference)
import functools

import jax
import jax.numpy as jnp
from jax import lax
from jax.experimental import pallas as pl
from jax.experimental.pallas import tpu as pltpu
from jax.experimental.pallas import tpu_sc as plsc

_L = 16
_S = 4096
_ROW = 50
_M = 1000000

_mesh = plsc.VectorSubcoreMesh(core_axis_name="c", subcore_axis_name="s")


def _log_f32(x):
    xi = plsc.bitcast(x, jnp.int32)
    e = (xi >> 23) - 127
    m = plsc.bitcast((xi & 0x7FFFFF) | (127 << 23), jnp.float32)
    t = (m - 1.0) / (m + 1.0)
    t2 = t * t
    poly = 1.0 + t2 * (1.0 / 3.0 + t2 * (1.0 / 5.0 + t2 * (1.0 / 7.0 + t2 / 9.0)))
    ln_m = 2.0 * t * poly
    return e.astype(jnp.float32) * 0.6931471805599453 + ln_m


@functools.partial(
    pl.kernel,
    out_type=jax.ShapeDtypeStruct((_L,), jnp.int32),
    mesh=_mesh,
    compiler_params=pltpu.CompilerParams(needs_layout_passes=False),
    cost_estimate=pl.CostEstimate(
        flops=10**8, transcendentals=10**6, bytes_accessed=10**8),
    scratch_types=[
        pltpu.VMEM((_L,), jnp.int32),
        pltpu.VMEM((_S,), jnp.float32),
        pltpu.VMEM((_L,), jnp.float32),
        pltpu.VMEM((_L,), jnp.int32),
        pltpu.VMEM((_L,), jnp.float32),
    ],
)
def _sc_scalars(combo_hbm, nlsd_hbm, len_ref, lp_ref, out_size_hbm,
                vc, vx, vlp, vwl, vwp):
    @pl.when(jnp.logical_and(lax.axis_index("c") == 0,
                             lax.axis_index("s") == 0))
    def _():
        pltpu.sync_copy(combo_hbm, vc)
        pltpu.sync_copy(nlsd_hbm, vx)

        lane = lax.iota(jnp.int32, _L)
        v = vc[...]
        zero = jnp.zeros((_L,), jnp.int32)
        idx = lax.reduce_max(jnp.where(lane < 8, v, zero), axes=(0,))
        vn = jnp.full((_L,), lax.reduce_max(jnp.where(lane >= 8, v, zero),
                                            axes=(0,)))

        base = pl.multiple_of(jnp.minimum((idx >> 3) << 3, _M - _L), 8)
        hit = lane == (idx - base)

        pltpu.sync_copy(len_ref.at[pl.ds(base, _L)], vwl)
        vwl[...] = jnp.where(hit, vn, vwl[...])
        pltpu.sync_copy(vwl, len_ref.at[pl.ds(base, _L)])

        def max_body(i, acc):
            return jnp.maximum(acc, vx[pl.ds(i * _L, _L)])

        mvec = lax.fori_loop(1, _S // _L, max_body, vx[pl.ds(0, _L)],
                             unroll=8)
        mmax = jnp.full((_L,), jnp.max(mvec))

        def sum_body(i, acc):
            return acc + jnp.exp(vx[pl.ds(i * _L, _L)] - mmax)

        svec = lax.fori_loop(0, _S // _L, sum_body,
                             jnp.zeros((_L,), jnp.float32), unroll=8)
        tot = jnp.full((_L,), jnp.sum(svec))
        vlp[...] = mmax + _log_f32(tot)

        pltpu.sync_copy(lp_ref.at[pl.ds(base, _L)], vwp)
        vwp[...] = jnp.where(hit, vlp[...], vwp[...])
        pltpu.sync_copy(vwp, lp_ref.at[pl.ds(base, _L)])

        vc[...] = v + 1
        pltpu.sync_copy(vc, out_size_hbm)


_BLK = 7936
_NBLK = (_M + _BLK - 1) // _BLK
_RB = (_ROW + 7) // 8


def _copy_body(size_ref, nseq_ref, in_ref, out_ref):
    out_ref[...] = in_ref[...]
    i = pl.program_id(1)
    idx = size_ref[0]

    @pl.when(i == idx // _BLK)
    def _():
        lb = pl.multiple_of(((idx % _BLK) >> 7) << 7, 128)
        gl = (i * _BLK + lb
              + lax.broadcasted_iota(jnp.int32, (8, 128), 1))
        win = out_ref[:, pl.ds(lb, 128)]
        out_ref[:, pl.ds(lb, 128)] = jnp.where(gl == idx, nseq_ref[...], win)


def _paste(size1, nseq_b, seq_t):
    return pl.pallas_call(
        _copy_body,
        grid_spec=pltpu.PrefetchScalarGridSpec(
            num_scalar_prefetch=1,
            grid=(_RB, _NBLK),
            in_specs=[
                pl.BlockSpec((8, 128), lambda j, i, sref: (j, 0)),
                pl.BlockSpec((8, _BLK), lambda j, i, sref: (j, i)),
            ],
            out_specs=pl.BlockSpec((8, _BLK), lambda j, i, sref: (j, i)),
        ),
        out_shape=jax.ShapeDtypeStruct((_ROW, _M), jnp.int32),
        compiler_params=pltpu.CompilerParams(
            dimension_semantics=("arbitrary", "arbitrary")),
    )(size1, nseq_b, seq_t)


def kernel(sequences, sequence_lengths, log_probabilities, size,
           node_sequence, node_sequence_length, node_log_state_distribution):
    size_i = jnp.asarray(size, jnp.int32)
    nsl_i = jnp.asarray(node_sequence_length, jnp.int32)
    combo = jnp.where(jnp.arange(_L) < 8, size_i, nsl_i)
    nseq_b = jnp.broadcast_to(
        jnp.pad(jnp.asarray(node_sequence, jnp.int32), (0, _RB * 8 - _ROW))
        [:, None], (_RB * 8, 128))
    size1 = size_i.reshape(1)

    seq_t = sequences.T
    len_ref = jax.new_ref(sequence_lengths)
    lp_ref = jax.new_ref(log_probabilities)

    out16 = _sc_scalars(combo, node_log_state_distribution, len_ref, lp_ref)
    seq_new_t = _paste(size1, nseq_b, seq_t)

    return seq_new_t.T, len_ref[...], lp_ref[...], out16[0]


# speedup vs baseline: 3.3889x; 3.3889x over previous
import functools

import jax
import jax.numpy as jnp
from jax import lax
from jax.experimental import pallas as pl
from jax.experimental.pallas import tpu as pltpu
from jax.experimental.pallas import tpu_sc as plsc

_L = 16
_S = 4096
_ROW = 50
_M = 1000000

_mesh = plsc.VectorSubcoreMesh(core_axis_name="c", subcore_axis_name="s")


def _log_f32(x):
    xi = plsc.bitcast(x, jnp.int32)
    e = (xi >> 23) - 127
    m = plsc.bitcast((xi & 0x7FFFFF) | (127 << 23), jnp.float32)
    t = (m - 1.0) / (m + 1.0)
    t2 = t * t
    poly = 1.0 + t2 * (1.0 / 3.0 + t2 * (1.0 / 5.0 + t2 * (1.0 / 7.0 + t2 / 9.0)))
    ln_m = 2.0 * t * poly
    return e.astype(jnp.float32) * 0.6931471805599453 + ln_m


@functools.partial(
    pl.kernel,
    out_type=jax.ShapeDtypeStruct((_L,), jnp.int32),
    mesh=_mesh,
    compiler_params=pltpu.CompilerParams(needs_layout_passes=False),
    cost_estimate=pl.CostEstimate(
        flops=10**8, transcendentals=10**6, bytes_accessed=10**8),
    scratch_types=[
        pltpu.VMEM((_L,), jnp.int32),
        pltpu.VMEM((_S,), jnp.float32),
        pltpu.VMEM((_L,), jnp.float32),
        pltpu.VMEM((_L,), jnp.int32),
        pltpu.VMEM((_L,), jnp.float32),
    ],
)
def _sc_scalars(combo_hbm, nlsd_hbm, len_ref, lp_ref, out_size_hbm,
                vc, vx, vlp, vwl, vwp):
    @pl.when(jnp.logical_and(lax.axis_index("c") == 0,
                             lax.axis_index("s") == 0))
    def _():
        pltpu.sync_copy(combo_hbm, vc)
        pltpu.sync_copy(nlsd_hbm, vx)

        lane = lax.iota(jnp.int32, _L)
        v = vc[...]
        zero = jnp.zeros((_L,), jnp.int32)
        idx = lax.reduce_max(jnp.where(lane < 8, v, zero), axes=(0,))
        vn = jnp.full((_L,), lax.reduce_max(jnp.where(lane >= 8, v, zero),
                                            axes=(0,)))

        base = pl.multiple_of(jnp.minimum((idx >> 3) << 3, _M - _L), 8)
        hit = lane == (idx - base)

        pltpu.sync_copy(len_ref.at[pl.ds(base, _L)], vwl)
        vwl[...] = jnp.where(hit, vn, vwl[...])
        pltpu.sync_copy(vwl, len_ref.at[pl.ds(base, _L)])

        def max_body(i, acc):
            return jnp.maximum(acc, vx[pl.ds(i * _L, _L)])

        mvec = lax.fori_loop(1, _S // _L, max_body, vx[pl.ds(0, _L)],
                             unroll=8)
        mmax = jnp.full((_L,), jnp.max(mvec))

        def sum_body(i, acc):
            return acc + jnp.exp(vx[pl.ds(i * _L, _L)] - mmax)

        svec = lax.fori_loop(0, _S // _L, sum_body,
                             jnp.zeros((_L,), jnp.float32), unroll=8)
        tot = jnp.full((_L,), jnp.sum(svec))
        vlp[...] = mmax + _log_f32(tot)

        pltpu.sync_copy(lp_ref.at[pl.ds(base, _L)], vwp)
        vwp[...] = jnp.where(hit, vlp[...], vwp[...])
        pltpu.sync_copy(vwp, lp_ref.at[pl.ds(base, _L)])

        vc[...] = v + 1
        pltpu.sync_copy(vc, out_size_hbm)


_RB = (_ROW + 7) // 8


def _paste_body(size_ref, nseq_ref, seq_win_ref, out_ref):
    idx = size_ref[0]
    base128 = (idx >> 7) << 7
    gl = base128 + lax.broadcasted_iota(jnp.int32, (_ROW, 128), 1)
    out_ref[...] = jnp.where(gl == idx, nseq_ref[...], seq_win_ref[...])


def _paste(size1, nseq_b, seq_t):
    return pl.pallas_call(
        _paste_body,
        grid_spec=pltpu.PrefetchScalarGridSpec(
            num_scalar_prefetch=1,
            grid=(1,),
            in_specs=[
                pl.BlockSpec((_ROW, 128), lambda i, sref: (0, 0)),
                pl.BlockSpec(
                    (_ROW, 128), lambda i, sref: (0, sref[0] // 128)),
            ],
            out_specs=pl.BlockSpec(
                (_ROW, 128), lambda i, sref: (0, sref[0] // 128)),
        ),
        out_shape=jax.ShapeDtypeStruct((_ROW, _M), jnp.int32),
        input_output_aliases={2: 0},
    )(size1, nseq_b, seq_t)


def kernel(sequences, sequence_lengths, log_probabilities, size,
           node_sequence, node_sequence_length, node_log_state_distribution):
    size_i = jnp.asarray(size, jnp.int32)
    nsl_i = jnp.asarray(node_sequence_length, jnp.int32)
    combo = jnp.where(jnp.arange(_L) < 8, size_i, nsl_i)
    nseq_b = jnp.broadcast_to(
        jnp.asarray(node_sequence, jnp.int32)[:, None], (_ROW, 128))
    size1 = size_i.reshape(1)

    seq_t = sequences.T
    len_ref = jax.new_ref(sequence_lengths)
    lp_ref = jax.new_ref(log_probabilities)

    out16 = _sc_scalars(combo, node_log_state_distribution, len_ref, lp_ref)
    seq_new_t = _paste(size1, nseq_b, seq_t)

    return seq_new_t.T, len_ref[...], lp_ref[...], out16[0]


# speedup vs baseline: 3.4192x; 1.0090x over previous
import functools

import jax
import jax.numpy as jnp
from jax import lax
from jax.experimental import pallas as pl
from jax.experimental.pallas import tpu as pltpu
from jax.experimental.pallas import tpu_sc as plsc

_L = 16
_S = 4096
_ROW = 50
_M = 1000000

_mesh = plsc.VectorSubcoreMesh(core_axis_name="c", subcore_axis_name="s")


def _log_f32(x):
    xi = plsc.bitcast(x, jnp.int32)
    e = (xi >> 23) - 127
    m = plsc.bitcast((xi & 0x7FFFFF) | (127 << 23), jnp.float32)
    t = (m - 1.0) / (m + 1.0)
    t2 = t * t
    poly = 1.0 + t2 * (1.0 / 3.0 + t2 * (1.0 / 5.0 + t2 * (1.0 / 7.0 + t2 / 9.0)))
    ln_m = 2.0 * t * poly
    return e.astype(jnp.float32) * 0.6931471805599453 + ln_m


@functools.partial(
    pl.kernel,
    out_type=jax.ShapeDtypeStruct((_L,), jnp.int32),
    mesh=_mesh,
    compiler_params=pltpu.CompilerParams(needs_layout_passes=False),
    cost_estimate=pl.CostEstimate(
        flops=10**8, transcendentals=10**6, bytes_accessed=10**8),
    scratch_types=[
        pltpu.VMEM((_L,), jnp.int32),
        pltpu.VMEM((_S,), jnp.float32),
        pltpu.VMEM((_L,), jnp.float32),
        pltpu.VMEM((_L,), jnp.int32),
        pltpu.VMEM((_L,), jnp.float32),
        pltpu.SemaphoreType.DMA,
        pltpu.SemaphoreType.DMA,
        pltpu.SemaphoreType.DMA,
        pltpu.SemaphoreType.DMA,
        pltpu.SemaphoreType.DMA,
    ],
)
def _sc_scalars(combo_hbm, nlsd_hbm, len_ref, lp_ref, out_size_hbm,
                vc, vx, vlp, vwl, vwp, s0, s1, s2, s3, s4):
    @pl.when(jnp.logical_and(lax.axis_index("c") == 0,
                             lax.axis_index("s") == 0))
    def _():
        c_combo = pltpu.async_copy(combo_hbm, vc, s0)
        c_nlsd = pltpu.async_copy(nlsd_hbm, vx, s1)
        c_combo.wait()

        lane = lax.iota(jnp.int32, _L)
        v = vc[...]
        zero = jnp.zeros((_L,), jnp.int32)
        idx = lax.reduce_max(jnp.where(lane < 8, v, zero), axes=(0,))
        vn = jnp.full((_L,), lax.reduce_max(jnp.where(lane >= 8, v, zero),
                                            axes=(0,)))

        base = pl.multiple_of(jnp.minimum((idx >> 3) << 3, _M - _L), 8)
        hit = lane == (idx - base)

        c_len = pltpu.async_copy(len_ref.at[pl.ds(base, _L)], vwl, s2)
        c_lp = pltpu.async_copy(lp_ref.at[pl.ds(base, _L)], vwp, s3)
        vc[...] = v + 1
        c_size = pltpu.async_copy(vc, out_size_hbm, s0)

        c_len.wait()
        vwl[...] = jnp.where(hit, vn, vwl[...])
        c_lenw = pltpu.async_copy(vwl, len_ref.at[pl.ds(base, _L)], s2)

        c_nlsd.wait()

        def max_body(i, acc):
            return jnp.maximum(acc, vx[pl.ds(i * _L, _L)])

        mvec = lax.fori_loop(1, _S // _L, max_body, vx[pl.ds(0, _L)],
                             unroll=8)
        mmax = jnp.full((_L,), jnp.max(mvec))

        def sum_body(i, acc):
            return acc + jnp.exp(vx[pl.ds(i * _L, _L)] - mmax)

        svec = lax.fori_loop(0, _S // _L, sum_body,
                             jnp.zeros((_L,), jnp.float32), unroll=8)
        tot = jnp.full((_L,), jnp.sum(svec))
        vlp[...] = mmax + _log_f32(tot)

        c_lp.wait()
        vwp[...] = jnp.where(hit, vlp[...], vwp[...])
        c_lpw = pltpu.async_copy(vwp, lp_ref.at[pl.ds(base, _L)], s4)

        c_size.wait()
        c_lenw.wait()
        c_lpw.wait()


_RB = (_ROW + 7) // 8


def _paste_body(size_ref, nseq_ref, seq_win_ref, out_ref):
    idx = size_ref[0]
    base128 = (idx >> 7) << 7
    gl = base128 + lax.broadcasted_iota(jnp.int32, (_ROW, 128), 1)
    out_ref[...] = jnp.where(gl == idx, nseq_ref[...], seq_win_ref[...])


def _paste(size1, nseq_b, seq_t):
    return pl.pallas_call(
        _paste_body,
        grid_spec=pltpu.PrefetchScalarGridSpec(
            num_scalar_prefetch=1,
            grid=(1,),
            in_specs=[
                pl.BlockSpec((_ROW, 128), lambda i, sref: (0, 0)),
                pl.BlockSpec(
                    (_ROW, 128), lambda i, sref: (0, sref[0] // 128)),
            ],
            out_specs=pl.BlockSpec(
                (_ROW, 128), lambda i, sref: (0, sref[0] // 128)),
        ),
        out_shape=jax.ShapeDtypeStruct((_ROW, _M), jnp.int32),
        input_output_aliases={2: 0},
    )(size1, nseq_b, seq_t)


def kernel(sequences, sequence_lengths, log_probabilities, size,
           node_sequence, node_sequence_length, node_log_state_distribution):
    size_i = jnp.asarray(size, jnp.int32)
    nsl_i = jnp.asarray(node_sequence_length, jnp.int32)
    combo = jnp.where(jnp.arange(_L) < 8, size_i, nsl_i)
    nseq_b = jnp.broadcast_to(
        jnp.asarray(node_sequence, jnp.int32)[:, None], (_ROW, 128))
    size1 = size_i.reshape(1)

    seq_t = sequences.T
    len_ref = jax.new_ref(sequence_lengths)
    lp_ref = jax.new_ref(log_probabilities)

    out16 = _sc_scalars(combo, node_log_state_distribution, len_ref, lp_ref)
    seq_new_t = _paste(size1, nseq_b, seq_t)

    return seq_new_t.T, len_ref[...], lp_ref[...], out16[0]
